# baseline (speedup 1.0000x reference)
import functools

import jax
import jax.numpy as jnp
from jax import lax
from jax.experimental import pallas as pl
from jax.experimental.pallas import tpu as pltpu
from jax.experimental.pallas import tpu_sc as plsc

GW = 128
NBUF = 4


def _tc_precompute(table_t, W, b):
    feat, n = table_t.shape
    blk = 2048

    def body(t_ref, w_ref, b_ref, o_ref):
        o_ref[...] = lax.dot_general(
            t_ref[...], w_ref[...], (((0,), (0,)), ((), ())),
            preferred_element_type=jnp.float32) + b_ref[...]

    return pl.pallas_call(
        body,
        grid=(n // blk,),
        in_specs=[
            pl.BlockSpec((feat, blk), lambda i: (0, i)),
            pl.BlockSpec((feat, 2 * feat), lambda i: (0, 0)),
            pl.BlockSpec((1, 2 * feat), lambda i: (0, 0)),
        ],
        out_specs=pl.BlockSpec((blk, 2 * feat), lambda i: (i, 0)),
        out_shape=jax.ShapeDtypeStruct((n, 2 * feat), jnp.float32),
    )(table_t, W, b)


def _sc_gather(t, idx2d):
    info = plsc.get_sparse_core_info()
    nc, ns = info.num_cores, info.num_subcores
    nw = nc * ns
    groups, gw = idx2d.shape
    width = t.shape[1]
    g_per_w = groups // nw
    mesh = plsc.VectorSubcoreMesh(core_axis_name="c", subcore_axis_name="s")

    @functools.partial(
        pl.kernel, mesh=mesh,
        out_type=jax.ShapeDtypeStruct((groups * gw, width), jnp.float32),
        scratch_types=[
            pltpu.VMEM((g_per_w, gw), jnp.int32),
            [pltpu.VMEM((gw, width), jnp.float32) for _ in range(NBUF)],
            [pltpu.SemaphoreType.DMA for _ in range(NBUF)],
        ],
    )
    def k(t_hbm, idx_hbm, out_hbm, idx_v, bufs, sems):
        wid = lax.axis_index("s") * nc + lax.axis_index("c")
        gbase = wid * g_per_w
        pltpu.sync_copy(idx_hbm.at[pl.ds(gbase, g_per_w)], idx_v)

        def start(j, b):
            pltpu.async_copy(t_hbm.at[idx_v.at[j]], bufs[b], sems[b])

        def finish(j, b):
            pltpu.make_async_copy(t_hbm.at[idx_v.at[j]], bufs[b],
                                  sems[b]).wait()
            pltpu.sync_copy(bufs[b], out_hbm.at[pl.ds((gbase + j) * gw, gw)])

        for b in range(NBUF):
            start(b, b)

        def body(j0, carry):
            for b in range(NBUF):
                j = j0 * NBUF + b
                finish(j, b)
                start(j + NBUF, b)
            return carry

        lax.fori_loop(0, g_per_w // NBUF - 1, body, 0)
        for b in range(NBUF):
            finish(g_per_w - NBUF + b, b)

    return k(t, idx2d)


def _tc_film_batch(g, xb, batch, nb, prev):
    _, feat, p = xb.shape
    blk = 2048
    jblocks = p // blk

    def body(g_ref, x_ref, *rest):
        o_ref = rest[-1]
        gv = g_ref[...]
        scale = jnp.transpose(gv[:, :feat])
        shift = jnp.transpose(gv[:, feat:])
        o_ref[0] = (x_ref[0] * scale) + shift

    in_specs = [
        pl.BlockSpec((blk, 2 * feat), lambda j: (j, 0)),
        pl.BlockSpec((1, feat, blk), lambda j: (0, 0, j)),
    ]
    operands = [g, xb]
    aliases = {}
    if prev is not None:
        in_specs.append(pl.BlockSpec(memory_space=pl.ANY))
        operands.append(prev)
        aliases = {2: 0}

    return pl.pallas_call(
        body,
        grid=(jblocks,),
        in_specs=in_specs,
        out_specs=pl.BlockSpec((1, feat, blk), lambda j: (batch, 0, j)),
        out_shape=jax.ShapeDtypeStruct((nb, feat, p), jnp.float32),
        input_output_aliases=aliases,
    )(*operands)


def kernel(x_zoom7, idx, group_idx, embeddings, W, b):
    nb, _, _, p, feat = x_zoom7.shape
    table_t = jnp.transpose(embeddings, (0, 2, 1))[0]
    t = _tc_precompute(table_t, W, b.reshape(1, -1))
    idx2d = idx.reshape(-1, GW)
    x3 = jnp.transpose(x_zoom7, (0, 1, 2, 4, 3)).reshape(nb, feat, p)
    gpb = idx2d.shape[0] // nb

    gathered = [
        _sc_gather(t, idx2d[bi * gpb:(bi + 1) * gpb]) for bi in range(nb)
    ]
    out3 = None
    for bi in range(nb):
        out3 = _tc_film_batch(gathered[bi], x3[bi:bi + 1], bi, nb, out3)
    return jnp.transpose(out3.reshape(nb, 1, 1, feat, p), (0, 1, 2, 4, 3))


# speedup vs baseline: 1.1217x; 1.1217x over previous
import functools

import jax
import jax.numpy as jnp
from jax import lax
from jax.experimental import pallas as pl
from jax.experimental.pallas import tpu as pltpu
from jax.experimental.pallas import tpu_sc as plsc

GW = 128
NBUF = 4


def _tc_precompute(table_t, W, b):
    feat, n = table_t.shape
    blk = 2048

    def body(t_ref, w_ref, b_ref, o_ref):
        o_ref[...] = lax.dot_general(
            t_ref[...], w_ref[...], (((0,), (0,)), ((), ())),
            preferred_element_type=jnp.float32) + b_ref[...]

    return pl.pallas_call(
        body,
        grid=(n // blk,),
        in_specs=[
            pl.BlockSpec((feat, blk), lambda i: (0, i)),
            pl.BlockSpec((feat, 2 * feat), lambda i: (0, 0)),
            pl.BlockSpec((1, 2 * feat), lambda i: (0, 0)),
        ],
        out_specs=pl.BlockSpec((blk, 2 * feat), lambda i: (i, 0)),
        out_shape=jax.ShapeDtypeStruct((n, 2 * feat), jnp.float32),
    )(table_t, W, b)


def _sc_gather(t, idx2d, base_group, chunk_groups):
    info = plsc.get_sparse_core_info()
    nc, ns = info.num_cores, info.num_subcores
    nw = nc * ns
    gw = idx2d.shape[1]
    width = t.shape[1]
    g_per_w = chunk_groups // nw
    mesh = plsc.VectorSubcoreMesh(core_axis_name="c", subcore_axis_name="s")

    @functools.partial(
        pl.kernel, mesh=mesh,
        out_type=jax.ShapeDtypeStruct((chunk_groups * gw, width), jnp.float32),
        scratch_types=[
            pltpu.VMEM((g_per_w, gw), jnp.int32),
            [pltpu.VMEM((gw, width), jnp.float32) for _ in range(NBUF)],
            [pltpu.SemaphoreType.DMA for _ in range(NBUF)],
        ],
    )
    def k(t_hbm, idx_hbm, out_hbm, idx_v, bufs, sems):
        wid = lax.axis_index("s") * nc + lax.axis_index("c")
        gbase = wid * g_per_w
        pltpu.sync_copy(idx_hbm.at[pl.ds(base_group + gbase, g_per_w)], idx_v)

        def start(j, b):
            pltpu.async_copy(t_hbm.at[idx_v.at[j]], bufs[b], sems[b])

        def finish(j, b):
            pltpu.make_async_copy(t_hbm.at[idx_v.at[j]], bufs[b],
                                  sems[b]).wait()
            pltpu.sync_copy(bufs[b], out_hbm.at[pl.ds((gbase + j) * gw, gw)])

        for b in range(NBUF):
            start(b, b)

        def body(j0, carry):
            for b in range(NBUF):
                j = j0 * NBUF + b
                finish(j, b)
                start(j + NBUF, b)
            return carry

        lax.fori_loop(0, g_per_w // NBUF - 1, body, 0)
        for b in range(NBUF):
            finish(g_per_w - NBUF + b, b)

    return k(t, idx2d)


def _tc_film_batch(g, x3, batch, prev):
    nb, feat, p = x3.shape
    blk = 2048
    jblocks = p // blk

    def body(g_ref, x_ref, *rest):
        o_ref = rest[-1]
        gv = g_ref[...]
        scale = jnp.transpose(gv[:, :feat])
        shift = jnp.transpose(gv[:, feat:])
        o_ref[0] = (x_ref[0] * scale) + shift

    in_specs = [
        pl.BlockSpec((blk, 2 * feat), lambda j: (j, 0)),
        pl.BlockSpec((1, feat, blk), lambda j: (batch, 0, j)),
    ]
    operands = [g, x3]
    aliases = {}
    if prev is not None:
        in_specs.append(pl.BlockSpec(memory_space=pl.ANY))
        operands.append(prev)
        aliases = {2: 0}

    return pl.pallas_call(
        body,
        grid=(jblocks,),
        in_specs=in_specs,
        out_specs=pl.BlockSpec((1, feat, blk), lambda j: (batch, 0, j)),
        out_shape=jax.ShapeDtypeStruct((nb, feat, p), jnp.float32),
        input_output_aliases=aliases,
    )(*operands)


def kernel(x_zoom7, idx, group_idx, embeddings, W, b):
    nb, _, _, p, feat = x_zoom7.shape
    table_t = jnp.transpose(embeddings, (0, 2, 1))[0]
    t = _tc_precompute(table_t, W, b.reshape(1, -1))
    idx2d = idx.reshape(-1, GW)
    x3 = jnp.transpose(x_zoom7, (0, 1, 2, 4, 3)).reshape(nb, feat, p)
    gpb = idx2d.shape[0] // nb

    gathered = [_sc_gather(t, idx2d, bi * gpb, gpb) for bi in range(nb)]
    out3 = None
    for bi in range(nb):
        out3 = _tc_film_batch(gathered[bi], x3, bi, out3)
    return jnp.transpose(out3.reshape(nb, 1, 1, feat, p), (0, 1, 2, 4, 3))
